# baseline (speedup 1.0000x reference)
import jax
import jax.numpy as jnp
from jax.experimental import pallas as pl
from jax.experimental.pallas import tpu as pltpu

_MASK_ID = 103
_TC_ROWS = 272


def kernel(inputs_embeds, word_emb, prompt_emb):
    B, S, H = inputs_embeds.shape
    P = prompt_emb.shape[0]
    T = 1 + P + S
    Tc = _TC_ROWS
    J = pl.cdiv(T, Tc)
    JI = pl.cdiv(S, Tc) - 1
    F = Tc - (1 + P)

    def body(in_ref, word_ref, prompt_ref, out_ref, carry_ref):
        j = pl.program_id(1)
        mr = _MASK_ID % 8

        @pl.when(j == 0)
        def _first():
            out_ref[0, pl.ds(0, 1), :] = word_ref[pl.ds(mr, 1), :]
            out_ref[0, pl.ds(1, P), :] = prompt_ref[...]

        @pl.when(j > 0)
        def _rest():
            out_ref[0, pl.ds(0, 1 + P), :] = carry_ref[...]

        out_ref[0, pl.ds(1 + P, F), :] = in_ref[0, pl.ds(0, F), :]
        carry_ref[...] = in_ref[0, pl.ds(F, 1 + P), :]

    return pl.pallas_call(
        body,
        grid=(B, J),
        in_specs=[
            pl.BlockSpec((1, Tc, H), lambda b, j: (b, jnp.minimum(j, JI), 0)),
            pl.BlockSpec((8, H), lambda b, j: (_MASK_ID // 8, 0)),
            pl.BlockSpec((P, H), lambda b, j: (0, 0)),
        ],
        out_specs=pl.BlockSpec((1, Tc, H), lambda b, j: (b, j, 0)),
        out_shape=jax.ShapeDtypeStruct((B, T, H), inputs_embeds.dtype),
        scratch_shapes=[pltpu.VMEM((1 + P, H), inputs_embeds.dtype)],
    )(inputs_embeds, word_emb, prompt_emb)


# speedup vs baseline: 3.6023x; 3.6023x over previous
import jax
import jax.numpy as jnp
from jax.experimental import pallas as pl
from jax.experimental.pallas import tpu as pltpu

_MASK_ID = 103
_TC_ROWS = 272


def kernel(inputs_embeds, word_emb, prompt_emb):
    B, S, H = inputs_embeds.shape
    P = prompt_emb.shape[0]
    T = 1 + P + S
    Tc = _TC_ROWS
    J = pl.cdiv(T, Tc)
    JI = pl.cdiv(S, Tc) - 1
    F = Tc - (1 + P)
    mr = _MASK_ID % 8

    def body(in_ref, word_ref, prompt_ref, out_ref, carry_ref):
        j = pl.program_id(0)

        @pl.when(j == 0)
        def _first():
            out_ref[pl.ds(0, 1), :, :] = jnp.broadcast_to(
                word_ref[pl.ds(mr, 1), :][:, None, :], (1, B, H)
            )
            out_ref[pl.ds(1, P), :, :] = jnp.broadcast_to(
                prompt_ref[...][:, None, :], (P, B, H)
            )

        @pl.when(j > 0)
        def _rest():
            out_ref[pl.ds(0, 1 + P), :, :] = carry_ref[...]

        out_ref[pl.ds(1 + P, F), :, :] = jnp.transpose(
            in_ref[:, pl.ds(0, F), :], (1, 0, 2)
        )
        carry_ref[...] = jnp.transpose(in_ref[:, pl.ds(F, 1 + P), :], (1, 0, 2))

    res = pl.pallas_call(
        body,
        grid=(J,),
        in_specs=[
            pl.BlockSpec((B, Tc, H), lambda j: (0, jnp.minimum(j, JI), 0)),
            pl.BlockSpec((8, H), lambda j: (_MASK_ID // 8, 0)),
            pl.BlockSpec((P, H), lambda j: (0, 0)),
        ],
        out_specs=pl.BlockSpec((Tc, B, H), lambda j: (j, 0, 0)),
        out_shape=jax.ShapeDtypeStruct((T, B, H), inputs_embeds.dtype),
        scratch_shapes=[pltpu.VMEM((1 + P, B, H), inputs_embeds.dtype)],
    )(inputs_embeds, word_emb, prompt_emb)
    return jnp.transpose(res, (1, 0, 2))


# speedup vs baseline: 3.6551x; 1.0147x over previous
import jax
import jax.numpy as jnp
from jax.experimental import pallas as pl
from jax.experimental.pallas import tpu as pltpu

_MASK_ID = 103
_TC_ROWS = 544


def kernel(inputs_embeds, word_emb, prompt_emb):
    B, S, H = inputs_embeds.shape
    P = prompt_emb.shape[0]
    T = 1 + P + S
    Tc = _TC_ROWS
    J = pl.cdiv(T, Tc)
    JI = pl.cdiv(S, Tc) - 1
    F = Tc - (1 + P)
    mr = _MASK_ID % 8

    def body(in_ref, word_ref, prompt_ref, out_ref, carry_ref):
        j = pl.program_id(0)

        @pl.when(j == 0)
        def _first():
            out_ref[pl.ds(0, 1), :, :] = jnp.broadcast_to(
                word_ref[pl.ds(mr, 1), :][:, None, :], (1, B, H)
            )
            out_ref[pl.ds(1, P), :, :] = jnp.broadcast_to(
                prompt_ref[...][:, None, :], (P, B, H)
            )

        @pl.when(j > 0)
        def _rest():
            out_ref[pl.ds(0, 1 + P), :, :] = carry_ref[...]

        out_ref[pl.ds(1 + P, F), :, :] = jnp.transpose(
            in_ref[:, pl.ds(0, F), :], (1, 0, 2)
        )
        carry_ref[...] = jnp.transpose(in_ref[:, pl.ds(F, 1 + P), :], (1, 0, 2))

    res = pl.pallas_call(
        body,
        grid=(J,),
        in_specs=[
            pl.BlockSpec((B, Tc, H), lambda j: (0, jnp.minimum(j, JI), 0)),
            pl.BlockSpec((8, H), lambda j: (_MASK_ID // 8, 0)),
            pl.BlockSpec((P, H), lambda j: (0, 0)),
        ],
        out_specs=pl.BlockSpec((Tc, B, H), lambda j: (j, 0, 0)),
        out_shape=jax.ShapeDtypeStruct((T, B, H), inputs_embeds.dtype),
        scratch_shapes=[pltpu.VMEM((1 + P, B, H), inputs_embeds.dtype)],
    )(inputs_embeds, word_emb, prompt_emb)
    return jnp.transpose(res, (1, 0, 2))
